# Initial kernel scaffold; baseline (speedup 1.0000x reference)
import jax
import jax.numpy as jnp
from jax.experimental import pallas as pl


def kernel(latents, embedding_weight):
    raise NotImplementedError("write your pallas kernel here")


# speedup vs baseline: 1.2051x; 1.2051x over previous
import jax
import jax.numpy as jnp
from jax import lax
from jax.experimental import pallas as pl
from jax.experimental.pallas import tpu as pltpu
from jax.experimental.pallas import tpu_sc as plsc

_K = 8192
_D = 256
_N = 8192
_TB = 256
_NT = _N // _TB

_NC, _NS = 2, 16
_NW = _NC * _NS
_BPW = _N // _NW
_CH = 128
_NCH = _BPW // _CH


def _dist_body(x_ref, et_ref, idx_ref, oh_ref, cnt_ref):
    i = pl.program_id(0)
    x = x_ref[...]
    et = et_ref[...]
    mm = jnp.dot(x, et, preferred_element_type=jnp.float32)
    x2 = jnp.sum(x * x, axis=1, keepdims=True)
    e2 = jnp.sum(et * et, axis=0, keepdims=True)
    d = (x2 + e2) - 2.0 * mm
    dmin = jnp.min(d, axis=1, keepdims=True)
    iota = lax.broadcasted_iota(jnp.int32, (_TB, _K), 1)
    cand = jnp.where(d == dmin, iota, _K)
    idx = jnp.min(cand, axis=1, keepdims=True)
    idx_ref[...] = idx
    oh = (iota == idx).astype(jnp.float32)
    oh_ref[...] = oh

    @pl.when(i == 0)
    def _():
        cnt_ref[...] = jnp.zeros_like(cnt_ref)

    cnt_ref[...] += jnp.sum(oh, axis=0, keepdims=True)


def _distances_argmin(flat, et):
    return pl.pallas_call(
        _dist_body,
        grid=(_NT,),
        in_specs=[
            pl.BlockSpec((_TB, _D), lambda i: (i, 0)),
            pl.BlockSpec((_D, _K), lambda i: (0, 0)),
        ],
        out_specs=[
            pl.BlockSpec((_TB, 1), lambda i: (i, 0)),
            pl.BlockSpec((_TB, _K), lambda i: (i, 0)),
            pl.BlockSpec((1, _K), lambda i: (0, 0)),
        ],
        out_shape=[
            jax.ShapeDtypeStruct((_N, 1), jnp.int32),
            jax.ShapeDtypeStruct((_N, _K), jnp.float32),
            jax.ShapeDtypeStruct((1, _K), jnp.float32),
        ],
        compiler_params=pltpu.CompilerParams(
            dimension_semantics=("arbitrary",),
        ),
    )(flat, et)


def _gather_body(table_hbm, idx_hbm, out_hbm, idx_v, rows_v, sem):
    wid = lax.axis_index("s") * _NC + lax.axis_index("c")
    pltpu.sync_copy(idx_hbm.at[pl.ds(wid * _NCH, _NCH)], idx_v)
    copies = [
        pltpu.async_copy(table_hbm.at[idx_v.at[j]],
                         rows_v.at[pl.ds(j * _CH, _CH)], sem)
        for j in range(_NCH)
    ]
    for c in copies:
        c.wait()
    pltpu.sync_copy(rows_v, out_hbm.at[pl.ds(wid * _BPW, _BPW)])


def _sc_gather(table, idx2d):
    fn = pl.kernel(
        _gather_body,
        mesh=plsc.VectorSubcoreMesh(core_axis_name="c", subcore_axis_name="s"),
        out_type=jax.ShapeDtypeStruct((_N, _D), jnp.float32),
        scratch_types=[
            pltpu.VMEM((_NCH, _CH), jnp.int32),
            pltpu.VMEM((_BPW, _D), jnp.float32),
            pltpu.SemaphoreType.DMA,
        ],
    )
    return fn(table, idx2d)


def _loss_body(q_ref, x_ref, cnt_ref, qst_ref, loss_ref, perp_ref):
    q = q_ref[...]
    x = x_ref[...]
    dqx = q - x
    qst_ref[...] = x + dqx
    mse = jnp.sum(dqx * dqx) * (1.0 / (_N * _D))
    loss_ref[...] = jnp.reshape(mse + 0.25 * mse, (1, 1))
    p = cnt_ref[...] * (1.0 / _N)
    ent = -jnp.sum(p * jnp.log(p + 1e-10))
    perp_ref[...] = jnp.reshape(jnp.exp(ent), (1, 1))


def _loss_perplexity(q, flat, counts):
    return pl.pallas_call(
        _loss_body,
        in_specs=[
            pl.BlockSpec((_N, _D), lambda: (0, 0)),
            pl.BlockSpec((_N, _D), lambda: (0, 0)),
            pl.BlockSpec((1, _K), lambda: (0, 0)),
        ],
        out_specs=[
            pl.BlockSpec((_N, _D), lambda: (0, 0)),
            pl.BlockSpec((1, 1), lambda: (0, 0)),
            pl.BlockSpec((1, 1), lambda: (0, 0)),
        ],
        out_shape=[
            jax.ShapeDtypeStruct((_N, _D), jnp.float32),
            jax.ShapeDtypeStruct((1, 1), jnp.float32),
            jax.ShapeDtypeStruct((1, 1), jnp.float32),
        ],
    )(q, flat, counts)


def kernel(latents, embedding_weight):
    lat = jnp.transpose(latents, (0, 2, 3, 1))
    flat = lat.reshape(_N, _D)
    et = embedding_weight.T
    idx, encodings, counts = _distances_argmin(flat, et)
    idx2d = idx.reshape(_N // _CH, _CH)
    q = _sc_gather(embedding_weight, idx2d)
    qst, loss, perp = _loss_perplexity(q, flat, counts)
    quantized_out = jnp.transpose(qst.reshape(8, 32, 32, _D), (0, 3, 1, 2))
    return (loss[0, 0], quantized_out, perp[0, 0], encodings)


# speedup vs baseline: 1.3324x; 1.1056x over previous
import jax
import jax.numpy as jnp
from jax import lax
from jax.experimental import pallas as pl
from jax.experimental.pallas import tpu as pltpu
from jax.experimental.pallas import tpu_sc as plsc

_K = 8192
_D = 256
_N = 8192
_TB = 256
_NT = _N // _TB

_NC, _NS = 2, 16
_NW = _NC * _NS
_BPW = _N // _NW
_CH = 128
_NCH = _BPW // _CH


def _dist_body(x_ref, e_ref, e2_ref, idx_ref, oh_ref, cnt_ref):
    i = pl.program_id(0)
    x = x_ref[...]
    xm2 = x * -2.0
    mm = lax.dot_general(xm2, e_ref[...], (((1,), (1,)), ((), ())),
                         preferred_element_type=jnp.float32)
    x2 = jnp.sum(x * x, axis=1, keepdims=True)
    d = (x2 + e2_ref[...]) + mm
    dmin = jnp.min(d, axis=1, keepdims=True)
    iota = lax.broadcasted_iota(jnp.int32, (_TB, _K), 1)
    idx = jnp.min(jnp.where(d == dmin, iota, _K),
                  axis=1, keepdims=True)
    idx_ref[...] = idx
    oh = (iota == idx).astype(jnp.float32)
    oh_ref[...] = oh

    @pl.when(i == 0)
    def _():
        cnt_ref[...] = jnp.zeros_like(cnt_ref)

    cnt_ref[...] += jnp.sum(oh, axis=0, keepdims=True)


def _distances_argmin(flat, emb, e2):
    return pl.pallas_call(
        _dist_body,
        grid=(_NT,),
        in_specs=[
            pl.BlockSpec((_TB, _D), lambda i: (i, 0)),
            pl.BlockSpec((_K, _D), lambda i: (0, 0)),
            pl.BlockSpec((1, _K), lambda i: (0, 0)),
        ],
        out_specs=[
            pl.BlockSpec((_TB, 1), lambda i: (i, 0)),
            pl.BlockSpec((_TB, _K), lambda i: (i, 0)),
            pl.BlockSpec((1, _K), lambda i: (0, 0)),
        ],
        out_shape=[
            jax.ShapeDtypeStruct((_N, 1), jnp.int32),
            jax.ShapeDtypeStruct((_N, _K), jnp.float32),
            jax.ShapeDtypeStruct((1, _K), jnp.float32),
        ],
        compiler_params=pltpu.CompilerParams(
            dimension_semantics=("arbitrary",),
        ),
    )(flat, emb, e2)


def _gather_body(table_hbm, idx_hbm, out_hbm, idx_v, rows_v, sem):
    wid = lax.axis_index("s") * _NC + lax.axis_index("c")
    pltpu.sync_copy(idx_hbm.at[pl.ds(wid * _NCH, _NCH)], idx_v)
    copies = [
        pltpu.async_copy(table_hbm.at[idx_v.at[j]],
                         rows_v.at[pl.ds(j * _CH, _CH)], sem)
        for j in range(_NCH)
    ]
    for c in copies:
        c.wait()
    pltpu.sync_copy(rows_v, out_hbm.at[pl.ds(wid * _BPW, _BPW)])


def _sc_gather(table, idx2d):
    fn = pl.kernel(
        _gather_body,
        mesh=plsc.VectorSubcoreMesh(core_axis_name="c", subcore_axis_name="s"),
        out_type=jax.ShapeDtypeStruct((_N, _D), jnp.float32),
        scratch_types=[
            pltpu.VMEM((_NCH, _CH), jnp.int32),
            pltpu.VMEM((_BPW, _D), jnp.float32),
            pltpu.SemaphoreType.DMA,
        ],
    )
    return fn(table, idx2d)


def _loss_body(q_ref, x_ref, cnt_ref, qst_ref, loss_ref, perp_ref):
    q = q_ref[...]
    x = x_ref[...]
    dqx = q - x
    qst_ref[...] = x + dqx
    mse = jnp.sum(dqx * dqx) * (1.0 / (_N * _D))
    loss_ref[...] = jnp.reshape(mse + 0.25 * mse, (1, 1))
    p = cnt_ref[...] * (1.0 / _N)
    ent = -jnp.sum(p * jnp.log(p + 1e-10))
    perp_ref[...] = jnp.reshape(jnp.exp(ent), (1, 1))


def _loss_perplexity(q, flat, counts):
    return pl.pallas_call(
        _loss_body,
        in_specs=[
            pl.BlockSpec((_N, _D), lambda: (0, 0)),
            pl.BlockSpec((_N, _D), lambda: (0, 0)),
            pl.BlockSpec((1, _K), lambda: (0, 0)),
        ],
        out_specs=[
            pl.BlockSpec((_N, _D), lambda: (0, 0)),
            pl.BlockSpec((1, 1), lambda: (0, 0)),
            pl.BlockSpec((1, 1), lambda: (0, 0)),
        ],
        out_shape=[
            jax.ShapeDtypeStruct((_N, _D), jnp.float32),
            jax.ShapeDtypeStruct((1, 1), jnp.float32),
            jax.ShapeDtypeStruct((1, 1), jnp.float32),
        ],
    )(q, flat, counts)


def kernel(latents, embedding_weight):
    lat = jnp.transpose(latents, (0, 2, 3, 1))
    flat = lat.reshape(_N, _D)
    e2 = jnp.sum(embedding_weight * embedding_weight, axis=1)[None, :]
    idx, encodings, counts = _distances_argmin(flat, embedding_weight, e2)
    idx2d = idx.reshape(_N // _CH, _CH)
    q = _sc_gather(embedding_weight, idx2d)
    qst, loss, perp = _loss_perplexity(q, flat, counts)
    quantized_out = jnp.transpose(qst.reshape(8, 32, 32, _D), (0, 3, 1, 2))
    return (loss[0, 0], quantized_out, perp[0, 0], encodings)


# speedup vs baseline: 1.5298x; 1.1482x over previous
import jax
import jax.numpy as jnp
from jax import lax
from jax.experimental import pallas as pl
from jax.experimental.pallas import tpu as pltpu
from jax.experimental.pallas import tpu_sc as plsc

_K = 8192
_D = 256
_N = 8192
_TB = 256
_NT = _N // _TB

_NC, _NS = 2, 16
_NW = _NC * _NS
_BPW = _N // _NW
_CH = 128
_NCH = _BPW // _CH


def _dist_body(x_ref, e_ref, e2_ref, idx_ref, oh_ref):
    x = x_ref[...]
    xm2 = x * -2.0
    mm = lax.dot_general(xm2, e_ref[...], (((1,), (1,)), ((), ())),
                         preferred_element_type=jnp.float32)
    x2 = jnp.sum(x * x, axis=1, keepdims=True)
    d = (x2 + e2_ref[...]) + mm
    dmin = jnp.min(d, axis=1, keepdims=True)
    iota = lax.broadcasted_iota(jnp.int32, (_TB, _K), 1)
    idx = jnp.min(jnp.where(d == dmin, iota, _K),
                  axis=1, keepdims=True)
    idx_ref[...] = idx
    oh = (iota == idx).astype(jnp.float32)
    oh_ref[...] = oh


def _distances_argmin(flat, emb, e2):
    return pl.pallas_call(
        _dist_body,
        grid=(_NT,),
        in_specs=[
            pl.BlockSpec((_TB, _D), lambda i: (i, 0)),
            pl.BlockSpec((_K, _D), lambda i: (0, 0)),
            pl.BlockSpec((1, _K), lambda i: (0, 0)),
        ],
        out_specs=[
            pl.BlockSpec((_TB, 1), lambda i: (i, 0)),
            pl.BlockSpec((_TB, _K), lambda i: (i, 0)),
        ],
        out_shape=[
            jax.ShapeDtypeStruct((_N, 1), jnp.int32),
            jax.ShapeDtypeStruct((_N, _K), jnp.float32),
        ],
        compiler_params=pltpu.CompilerParams(
            dimension_semantics=("arbitrary",),
        ),
    )(flat, emb, e2)


def _gather_body(table_hbm, idx_hbm, zeros_hbm, out_hbm, cnt_hbm,
                 idx_v, rows_v, ones_v, cnt_sh, sem):
    c = lax.axis_index("c")
    s = lax.axis_index("s")
    wid = s * _NC + c
    pltpu.sync_copy(idx_hbm.at[pl.ds(wid * _NCH, _NCH)], idx_v)
    copies = [
        pltpu.async_copy(table_hbm.at[idx_v.at[j]],
                         rows_v.at[pl.ds(j * _CH, _CH)], sem)
        for j in range(_NCH)
    ]
    @pl.when(s == 0)
    def _():
        pltpu.sync_copy(zeros_hbm, cnt_sh)
    for j in range(_CH // 16):
        ones_v[pl.ds(j * 16, 16)] = jnp.full((16,), 1.0, jnp.float32)
    plsc.subcore_barrier()
    for j in range(_NCH):
        pltpu.sync_copy(ones_v, cnt_sh.at[idx_v.at[j]], add=True)
    for cp in copies:
        cp.wait()
    pltpu.sync_copy(rows_v, out_hbm.at[pl.ds(wid * _BPW, _BPW)])
    plsc.subcore_barrier()
    @pl.when(s == 0)
    def _():
        pltpu.sync_copy(cnt_sh, cnt_hbm.at[c])


def _sc_gather_hist(table, idx2d, zeros_k):
    fn = pl.kernel(
        _gather_body,
        mesh=plsc.VectorSubcoreMesh(core_axis_name="c", subcore_axis_name="s"),
        out_type=[
            jax.ShapeDtypeStruct((_N, _D), jnp.float32),
            jax.ShapeDtypeStruct((_NC, _K), jnp.float32),
        ],
        scratch_types=[
            pltpu.VMEM((_NCH, _CH), jnp.int32),
            pltpu.VMEM((_BPW, _D), jnp.float32),
            pltpu.VMEM((_CH,), jnp.float32),
            pltpu.VMEM_SHARED((_K,), jnp.float32),
            pltpu.SemaphoreType.DMA,
        ],
    )
    return fn(table, idx2d, zeros_k)


def _loss_body(q_ref, x_ref, cnt_ref, qst_ref, loss_ref, perp_ref):
    q = q_ref[...]
    x = x_ref[...]
    dqx = q - x
    qst_ref[...] = x + dqx
    mse = jnp.sum(dqx * dqx) * (1.0 / (_N * _D))
    loss_ref[...] = jnp.reshape(mse + 0.25 * mse, (1, 1))
    cnt = cnt_ref[...]
    p = (cnt[0:1, :] + cnt[1:2, :]) * (1.0 / _N)
    ent = -jnp.sum(p * jnp.log(p + 1e-10))
    perp_ref[...] = jnp.reshape(jnp.exp(ent), (1, 1))


def _loss_perplexity(q, flat, counts):
    return pl.pallas_call(
        _loss_body,
        in_specs=[
            pl.BlockSpec((_N, _D), lambda: (0, 0)),
            pl.BlockSpec((_N, _D), lambda: (0, 0)),
            pl.BlockSpec((_NC, _K), lambda: (0, 0)),
        ],
        out_specs=[
            pl.BlockSpec((_N, _D), lambda: (0, 0)),
            pl.BlockSpec((1, 1), lambda: (0, 0)),
            pl.BlockSpec((1, 1), lambda: (0, 0)),
        ],
        out_shape=[
            jax.ShapeDtypeStruct((_N, _D), jnp.float32),
            jax.ShapeDtypeStruct((1, 1), jnp.float32),
            jax.ShapeDtypeStruct((1, 1), jnp.float32),
        ],
    )(q, flat, counts)


def kernel(latents, embedding_weight):
    lat = jnp.transpose(latents, (0, 2, 3, 1))
    flat = lat.reshape(_N, _D)
    e2 = jnp.sum(embedding_weight * embedding_weight, axis=1)[None, :]
    idx, encodings = _distances_argmin(flat, embedding_weight, e2)
    idx2d = idx.reshape(_N // _CH, _CH)
    zeros_k = jnp.zeros((_K,), jnp.float32)
    q, counts = _sc_gather_hist(embedding_weight, idx2d, zeros_k)
    qst, loss, perp = _loss_perplexity(q, flat, counts)
    quantized_out = jnp.transpose(qst.reshape(8, 32, 32, _D), (0, 3, 1, 2))
    return (loss[0, 0], quantized_out, perp[0, 0], encodings)
